# baseline (speedup 1.0000x reference)
import functools

import jax
import jax.numpy as jnp
from jax import lax
from jax.experimental import pallas as pl
from jax.experimental.pallas import tpu as pltpu
from jax.experimental.pallas import tpu_sc as plsc

NC = 2
NS = 16
L = 16
NBUF = 5
HP = 128
BLKV = 16384


def _pad_transpose_tc(embT):
    H, V = embT.shape
    grid = (V + BLKV - 1) // BLKV

    def body(in_ref, out_ref):
        y = jnp.swapaxes(in_ref[...], 0, 1)
        z = jnp.zeros((BLKV, HP - H), jnp.float32)
        out_ref[...] = jnp.concatenate([y, z], axis=1)

    return pl.pallas_call(
        body,
        out_shape=jax.ShapeDtypeStruct((V, HP), jnp.float32),
        grid=(grid,),
        in_specs=[pl.BlockSpec((H, BLKV), lambda j: (0, j))],
        out_specs=pl.BlockSpec((BLKV, HP), lambda j: (j, 0)),
    )(embT)


def kernel(word_inputs, embedding):
    B, S = word_inputs.shape
    V, H = embedding.shape
    NW = NC * NS
    BPW = B // NW
    assert S % NBUF == 0

    mesh = plsc.VectorSubcoreMesh(
        core_axis_name="c", subcore_axis_name="s", num_cores=NC, num_subcores=NS
    )

    @functools.partial(
        pl.kernel,
        out_type=jax.ShapeDtypeStruct((S * B, HP), jnp.float32),
        mesh=mesh,
        scratch_types=(
            [pltpu.VMEM((S, BPW), jnp.int32)]
            + [pltpu.VMEM((BPW, HP), jnp.float32)] * NBUF
            + [pltpu.SemaphoreType.DMA] * (2 * NBUF)
        ),
    )
    def emb_lookup(wordT_hbm, emb_hbm, out_hbm, idx_v, *bufs):
        rows = bufs[:NBUF]
        gsem = bufs[NBUF : 2 * NBUF]
        wsem = bufs[2 * NBUF : 3 * NBUF]

        cid = lax.axis_index("c")
        sid = lax.axis_index("s")
        wid = sid * NC + cid
        b0 = wid * BPW
        pltpu.sync_copy(wordT_hbm.at[:, pl.ds(b0, BPW)], idx_v)

        def wait_gather(b):
            pltpu.make_async_copy(emb_hbm.at[idx_v.at[0]], rows[b], gsem[b]).wait()

        def fire_write(i, b):
            pltpu.async_copy(
                rows[b], out_hbm.at[pl.ds(i * B + b0, BPW), :], wsem[b]
            )

        def wait_write(b):
            pltpu.make_async_copy(
                rows[b], out_hbm.at[pl.ds(b0, BPW), :], wsem[b]
            ).wait()

        @pl.loop(0, S, step=NBUF)
        def _(g):
            for b in range(NBUF):
                i = g + b

                @pl.when(g + b >= NBUF)
                def _():
                    wait_write(b)

                pltpu.async_copy(emb_hbm.at[idx_v.at[i]], rows[b], gsem[b])
                pb = (b - 2) % NBUF

                @pl.when(g + b >= 2)
                def _():
                    wait_gather(pb)
                    fire_write(i - 2, pb)

        for k in (2, 1):
            lastk = (S - k) % NBUF
            wait_gather(lastk)
            fire_write(S - k, lastk)
        for b in range(NBUF):
            wait_write(b)

    wordT = jnp.transpose(word_inputs.astype(jnp.int32))
    emb_padded = _pad_transpose_tc(jnp.transpose(embedding))
    out = emb_lookup(wordT, emb_padded)
    return out[:, :H].reshape(S, B, H)


# speedup vs baseline: 1.0130x; 1.0130x over previous
import functools

import jax
import jax.numpy as jnp
from jax import lax
from jax.experimental import pallas as pl
from jax.experimental.pallas import tpu as pltpu
from jax.experimental.pallas import tpu_sc as plsc

NC = 2
NS = 16
L = 16
NBUF = 4
HP = 128
BLKV = 32768


def _pad_transpose_tc(embT):
    H, V = embT.shape
    grid = (V + BLKV - 1) // BLKV

    def body(in_ref, out_ref):
        y = jnp.swapaxes(in_ref[...], 0, 1)
        z = jnp.zeros((BLKV, HP - H), jnp.float32)
        out_ref[...] = jnp.concatenate([y, z], axis=1)

    return pl.pallas_call(
        body,
        out_shape=jax.ShapeDtypeStruct((V, HP), jnp.float32),
        grid=(grid,),
        in_specs=[pl.BlockSpec((H, BLKV), lambda j: (0, j))],
        out_specs=pl.BlockSpec((BLKV, HP), lambda j: (j, 0)),
    )(embT)


def kernel(word_inputs, embedding):
    B, S = word_inputs.shape
    V, H = embedding.shape
    NW = NC * NS
    BPW = B // NW
    assert S % NBUF == 0

    mesh = plsc.VectorSubcoreMesh(
        core_axis_name="c", subcore_axis_name="s", num_cores=NC, num_subcores=NS
    )

    @functools.partial(
        pl.kernel,
        out_type=jax.ShapeDtypeStruct((S * B, HP), jnp.float32),
        mesh=mesh,
        scratch_types=(
            [pltpu.VMEM((S, BPW), jnp.int32)]
            + [pltpu.VMEM((BPW, HP), jnp.float32)] * NBUF
            + [pltpu.SemaphoreType.DMA] * (2 * NBUF)
        ),
    )
    def emb_lookup(wordT_hbm, emb_hbm, out_hbm, idx_v, *bufs):
        rows = bufs[:NBUF]
        gsem = bufs[NBUF : 2 * NBUF]
        wsem = bufs[2 * NBUF : 3 * NBUF]

        cid = lax.axis_index("c")
        sid = lax.axis_index("s")
        wid = sid * NC + cid
        b0 = wid * BPW
        pltpu.sync_copy(wordT_hbm.at[:, pl.ds(b0, BPW)], idx_v)

        def wait_gather(b):
            pltpu.make_async_copy(emb_hbm.at[idx_v.at[0]], rows[b], gsem[b]).wait()

        def fire_write(i, b):
            pltpu.async_copy(
                rows[b], out_hbm.at[pl.ds(i * B + b0, BPW), :], wsem[b]
            )

        def wait_write(b):
            pltpu.make_async_copy(
                rows[b], out_hbm.at[pl.ds(b0, BPW), :], wsem[b]
            ).wait()

        @pl.loop(0, S, step=NBUF)
        def _(g):
            for b in range(NBUF):
                i = g + b

                @pl.when(g + b >= NBUF)
                def _():
                    wait_write(b)

                pltpu.async_copy(emb_hbm.at[idx_v.at[i]], rows[b], gsem[b])
                pb = (b - 1) % NBUF

                @pl.when(g + b >= 1)
                def _():
                    wait_gather(pb)
                    fire_write(i - 1, pb)

        last = (S - 1) % NBUF
        wait_gather(last)
        fire_write(S - 1, last)
        for b in range(NBUF):
            wait_write(b)

    wordT = jnp.transpose(word_inputs.astype(jnp.int32))
    emb_padded = _pad_transpose_tc(jnp.transpose(embedding))
    out = emb_lookup(wordT, emb_padded)
    return out[:, :H].reshape(S, B, H)
